# initial kernel scaffold (unmeasured)
import jax
import jax.numpy as jnp
from jax import lax
from jax.experimental import pallas as pl
from jax.experimental.pallas import tpu as pltpu

B, H, D, BS = 16, 16, 64, 16
NPAGES_LOCAL = 128
NKEYS = NPAGES_LOCAL * BS
NSLOTS = 128
SCALE = D ** -0.5
NEG = -1e30


def _body(q_ref, k_ref, v_ref, bt_ref, lens_ref, out_ref,
          m_snd, l_snd, o_snd, m_rcv, l_rcv, o_rcv,
          send_sems, recv_sems):
    my_x = lax.axis_index("x")
    my_y = lax.axis_index("y")
    peer = (my_x, 1 - my_y)

    barrier = pltpu.get_barrier_semaphore()
    pl.semaphore_signal(barrier, inc=1, device_id=peer,
                        device_id_type=pl.DeviceIdType.MESH)
    pl.semaphore_wait(barrier, 1)

    bt = bt_ref[...] - my_y * NPAGES_LOCAL
    iota_j = lax.broadcasted_iota(jnp.int32, (B, NSLOTS), 1)
    valid = iota_j < lens_ref[...]
    iota_p = lax.broadcasted_iota(jnp.int32, (B, NSLOTS, NPAGES_LOCAL), 2)
    hits = (bt[:, :, None] == iota_p) & valid[:, :, None]
    counts = jnp.sum(hits.astype(jnp.float32), axis=1)

    expand = (
        lax.broadcasted_iota(jnp.int32, (NPAGES_LOCAL, NKEYS), 0)
        == lax.broadcasted_iota(jnp.int32, (NPAGES_LOCAL, NKEYS), 1) // BS
    ).astype(jnp.float32)
    w = jax.lax.dot_general(counts, expand, (((1,), (0,)), ((), ())),
                            preferred_element_type=jnp.float32)
    wpos = w > 0.0

    for h in range(H):
        qh = q_ref[:, h, :]
        kh = k_ref[:, h, :]
        s = jax.lax.dot_general(qh, kh, (((1,), (1,)), ((), ())),
                                preferred_element_type=jnp.float32) * SCALE
        s = jnp.where(wpos, s, NEG)
        m_h = jnp.max(s, axis=1, keepdims=True)
        e = w * jnp.exp(s - m_h)
        l_h = jnp.sum(e, axis=1, keepdims=True)
        vh = v_ref[:, h, :]
        o_h = jax.lax.dot_general(e, vh, (((1,), (0,)), ((), ())),
                                  preferred_element_type=jnp.float32)
        m_snd[:, h:h + 1] = m_h
        l_snd[:, h:h + 1] = l_h
        o_snd[:, h, :] = o_h

    copies = [
        pltpu.make_async_remote_copy(
            src_ref=src, dst_ref=dst,
            send_sem=send_sems.at[i], recv_sem=recv_sems.at[i],
            device_id=peer, device_id_type=pl.DeviceIdType.MESH)
        for i, (src, dst) in enumerate(
            [(m_snd, m_rcv), (l_snd, l_rcv), (o_snd, o_rcv)])
    ]
    for c in copies:
        c.start()
    for c in copies:
        c.wait()

    m_s, m_r = m_snd[...], m_rcv[...]
    m_n = jnp.maximum(m_s, m_r)
    a_s = jnp.exp(m_s - m_n)
    a_r = jnp.exp(m_r - m_n)
    l_n = l_snd[...] * a_s + l_rcv[...] * a_r
    o_n = o_snd[...] * a_s[:, :, None] + o_rcv[...] * a_r[:, :, None]
    out_ref[...] = o_n / l_n[:, :, None]


def kernel(Q, K, V, bt, lens):
    q = Q.reshape(B, H, D)
    k = K.reshape(NKEYS, H, D)
    v = V.reshape(NKEYS, H, D)
    lens2 = lens.reshape(B, 1)
    out = pl.pallas_call(
        _body,
        out_shape=jax.ShapeDtypeStruct((B, H, D), jnp.float32),
        in_specs=[pl.BlockSpec(memory_space=pltpu.VMEM)] * 5,
        out_specs=pl.BlockSpec(memory_space=pltpu.VMEM),
        scratch_shapes=[
            pltpu.VMEM((B, H), jnp.float32),
            pltpu.VMEM((B, H), jnp.float32),
            pltpu.VMEM((B, H, D), jnp.float32),
            pltpu.VMEM((B, H), jnp.float32),
            pltpu.VMEM((B, H), jnp.float32),
            pltpu.VMEM((B, H, D), jnp.float32),
            pltpu.SemaphoreType.DMA((3,)),
            pltpu.SemaphoreType.DMA((3,)),
        ],
        compiler_params=pltpu.CompilerParams(collective_id=0),
    )(q, k, v, bt, lens2)
    return out.reshape(B, 1, H, D)


# baseline (device time: 65162 ns/iter reference)
import jax
import jax.numpy as jnp
from jax import lax
from jax.experimental import pallas as pl
from jax.experimental.pallas import tpu as pltpu

B, H, D, BS = 16, 16, 64, 16
NPAGES_LOCAL = 128
NKEYS = NPAGES_LOCAL * BS
NSLOTS = 128
SCALE = D ** -0.5
NEG = -1e30


def _body(q_ref, k_ref, v_ref, bt_ref, lens_ref, out_ref,
          m_snd, l_snd, o_snd, m_rcv, l_rcv, o_rcv,
          send_sems, recv_sems):
    my_x = lax.axis_index("x")
    my_y = lax.axis_index("y")
    peer = (my_x, 1 - my_y)

    barrier = pltpu.get_barrier_semaphore()
    pl.semaphore_signal(barrier, inc=1, device_id=peer,
                        device_id_type=pl.DeviceIdType.MESH)
    pl.semaphore_wait(barrier, 1)

    bt = bt_ref[...] - my_y * NPAGES_LOCAL
    iota_j = lax.broadcasted_iota(jnp.int32, (B, NSLOTS), 1)
    valid = iota_j < lens_ref[...]
    btv = jnp.where(valid, bt, -1)
    iota_p = lax.broadcasted_iota(jnp.int32, (B, NPAGES_LOCAL, NSLOTS), 1)
    hits = btv[:, None, :] == iota_p
    counts = jnp.sum(hits.astype(jnp.float32), axis=2)

    expand = (
        lax.broadcasted_iota(jnp.int32, (NPAGES_LOCAL, NKEYS), 0)
        == lax.broadcasted_iota(jnp.int32, (NPAGES_LOCAL, NKEYS), 1) // BS
    ).astype(jnp.float32)
    w = jax.lax.dot_general(counts, expand, (((1,), (0,)), ((), ())),
                            preferred_element_type=jnp.float32)
    wpos = w > 0.0

    for h in range(H):
        qh = q_ref[:, h, :]
        kh = k_ref[:, h, :]
        s = jax.lax.dot_general(qh, kh, (((1,), (1,)), ((), ())),
                                preferred_element_type=jnp.float32) * SCALE
        s = jnp.where(wpos, s, NEG)
        m_h = jnp.max(s, axis=1, keepdims=True)
        e = w * jnp.exp(s - m_h)
        l_h = jnp.sum(e, axis=1, keepdims=True)
        vh = v_ref[:, h, :]
        o_h = jax.lax.dot_general(e, vh, (((1,), (0,)), ((), ())),
                                  preferred_element_type=jnp.float32)
        m_snd[:, h:h + 1] = m_h
        l_snd[:, h:h + 1] = l_h
        o_snd[:, h, :] = o_h

    copies = [
        pltpu.make_async_remote_copy(
            src_ref=src, dst_ref=dst,
            send_sem=send_sems.at[i], recv_sem=recv_sems.at[i],
            device_id=peer, device_id_type=pl.DeviceIdType.MESH)
        for i, (src, dst) in enumerate(
            [(m_snd, m_rcv), (l_snd, l_rcv), (o_snd, o_rcv)])
    ]
    for c in copies:
        c.start()
    for c in copies:
        c.wait()

    m_s, m_r = m_snd[...], m_rcv[...]
    m_n = jnp.maximum(m_s, m_r)
    a_s = jnp.exp(m_s - m_n)
    a_r = jnp.exp(m_r - m_n)
    l_n = l_snd[...] * a_s + l_rcv[...] * a_r
    for h in range(H):
        o_h = (o_snd[:, h, :] * a_s[:, h:h + 1]
               + o_rcv[:, h, :] * a_r[:, h:h + 1])
        out_ref[:, h, :] = o_h / l_n[:, h:h + 1]


def kernel(Q, K, V, bt, lens):
    q = Q.reshape(B, H, D)
    k = K.reshape(NKEYS, H, D)
    v = V.reshape(NKEYS, H, D)
    lens2 = lens.reshape(B, 1)
    out = pl.pallas_call(
        _body,
        out_shape=jax.ShapeDtypeStruct((B, H, D), jnp.float32),
        in_specs=[pl.BlockSpec(memory_space=pltpu.VMEM)] * 5,
        out_specs=pl.BlockSpec(memory_space=pltpu.VMEM),
        scratch_shapes=[
            pltpu.VMEM((B, H), jnp.float32),
            pltpu.VMEM((B, H), jnp.float32),
            pltpu.VMEM((B, H, D), jnp.float32),
            pltpu.VMEM((B, H), jnp.float32),
            pltpu.VMEM((B, H), jnp.float32),
            pltpu.VMEM((B, H, D), jnp.float32),
            pltpu.SemaphoreType.DMA((3,)),
            pltpu.SemaphoreType.DMA((3,)),
        ],
        compiler_params=pltpu.CompilerParams(collective_id=0),
    )(q, k, v, bt, lens2)
    return out.reshape(B, 1, H, D)
